# baseline (speedup 1.0000x reference)
import functools

import jax
import jax.numpy as jnp
from jax import lax
from jax.experimental import pallas as pl
from jax.experimental.pallas import tpu as pltpu
from jax.experimental.pallas import tpu_sc as plsc

SENT = 200
EMB = 128
NCLS = 100000
BLK = 25000
NB = NCLS // BLK
CHUNK = 8
NWORK = SENT // CHUNK


def _sc_gather_sum(idx_hbm, table_hbm, out_hbm, idx_v, rows_v, acc_v, sem):
    wid = lax.axis_index("s") * 2 + lax.axis_index("c")

    @pl.when(wid < NWORK)
    def _():
        pltpu.sync_copy(idx_hbm.at[pl.ds(wid * CHUNK, CHUNK)], idx_v)
        pltpu.async_copy(table_hbm.at[idx_v], rows_v, sem).wait()
        for d in range(EMB // 16):
            v = rows_v[0, pl.ds(d * 16, 16)]
            for r in range(1, CHUNK):
                v = v + rows_v[r, pl.ds(d * 16, 16)]
            acc_v[0, pl.ds(d * 16, 16)] = v
        pltpu.sync_copy(acc_v, out_hbm.at[pl.ds(wid, 1)])


_gather_sum = functools.partial(
    pl.kernel,
    mesh=plsc.VectorSubcoreMesh(core_axis_name="c", subcore_axis_name="s"),
    out_type=jax.ShapeDtypeStruct((NWORK, EMB), jnp.float32),
    scratch_types=[
        pltpu.VMEM((CHUNK,), jnp.int32),
        pltpu.VMEM((CHUNK, EMB), jnp.float32),
        pltpu.VMEM((1, EMB), jnp.float32),
        pltpu.SemaphoreType.DMA,
    ],
)(_sc_gather_sum)


def _tc_matvec_lse(part_ref, w_ref, b_ref, out_ref):
    j = pl.program_id(0)
    e = jnp.sum(part_ref[...], axis=0, keepdims=True)
    logits = lax.dot_general(
        e, w_ref[...], (((1,), (1,)), ((), ())),
        preferred_element_type=jnp.float32,
    ) + b_ref[pl.ds(j, 1), :]
    out_ref[pl.ds(j, 1), :] = logits

    @pl.when(j == NB - 1)
    def _():
        whole = out_ref[...]
        m = jnp.max(whole)
        s = jnp.sum(jnp.exp(whole - m))
        out_ref[...] = whole - (m + jnp.log(s))


def kernel(indices, emb_table, W0, b0, W1, b1, W2, b2):
    del W0, b0, W1, b1
    idx = indices.astype(jnp.int32)
    partials = _gather_sum(idx, emb_table)
    out = pl.pallas_call(
        _tc_matvec_lse,
        grid=(NB,),
        in_specs=[
            pl.BlockSpec((NWORK, EMB), lambda j: (0, 0)),
            pl.BlockSpec((BLK, EMB), lambda j: (j, 0)),
            pl.BlockSpec((NB, BLK), lambda j: (0, 0)),
        ],
        out_specs=pl.BlockSpec((NB, BLK), lambda j: (0, 0)),
        out_shape=jax.ShapeDtypeStruct((NB, BLK), jnp.float32),
    )(partials, W2, b2.reshape(NB, BLK))
    return out.reshape(1, NCLS)


# speedup vs baseline: 1.0149x; 1.0149x over previous
import functools

import jax
import jax.numpy as jnp
from jax import lax
from jax.experimental import pallas as pl
from jax.experimental.pallas import tpu as pltpu
from jax.experimental.pallas import tpu_sc as plsc

SENT = 200
EMB = 128
NCLS = 100000
BLK = 20000
NB = NCLS // BLK
CHUNK = 8
NWORK = SENT // CHUNK


def _sc_gather_sum(idx_hbm, table_hbm, out_hbm, idx_v, rows_v, acc_v, sem):
    wid = lax.axis_index("s") * 2 + lax.axis_index("c")

    @pl.when(wid < NWORK)
    def _():
        pltpu.sync_copy(idx_hbm.at[pl.ds(wid * CHUNK, CHUNK)], idx_v)
        pltpu.async_copy(table_hbm.at[idx_v], rows_v, sem).wait()
        for d in range(EMB // 16):
            v = rows_v[0, pl.ds(d * 16, 16)]
            for r in range(1, CHUNK):
                v = v + rows_v[r, pl.ds(d * 16, 16)]
            acc_v[0, pl.ds(d * 16, 16)] = v
        pltpu.sync_copy(acc_v, out_hbm.at[pl.ds(wid, 1)])


_gather_sum = functools.partial(
    pl.kernel,
    mesh=plsc.VectorSubcoreMesh(core_axis_name="c", subcore_axis_name="s"),
    out_type=jax.ShapeDtypeStruct((NWORK, EMB), jnp.float32),
    scratch_types=[
        pltpu.VMEM((CHUNK,), jnp.int32),
        pltpu.VMEM((CHUNK, EMB), jnp.float32),
        pltpu.VMEM((1, EMB), jnp.float32),
        pltpu.SemaphoreType.DMA,
    ],
)(_sc_gather_sum)


def _tc_matvec_lse(part_ref, w_ref, b_ref, out_ref, mv_ref, sv_ref):
    j = pl.program_id(0)
    e = jnp.sum(part_ref[...], axis=0, keepdims=True)
    logits = lax.dot_general(
        e, w_ref[...], (((1,), (1,)), ((), ())),
        preferred_element_type=jnp.float32,
    ) + b_ref[pl.ds(j, 1), :]
    out_ref[pl.ds(j, 1), :] = logits

    @pl.when(j == 0)
    def _():
        mv_ref[...] = jnp.full((1, BLK), -jnp.inf, jnp.float32)
        sv_ref[...] = jnp.zeros((1, BLK), jnp.float32)

    mv_old = mv_ref[...]
    mv_new = jnp.maximum(mv_old, logits)
    sv_ref[...] = (sv_ref[...] * jnp.exp(mv_old - mv_new)
                   + jnp.exp(logits - mv_new))
    mv_ref[...] = mv_new

    @pl.when(j == NB - 1)
    def _():
        mv = mv_ref[...]
        m = jnp.max(mv)
        s = jnp.sum(sv_ref[...] * jnp.exp(mv - m))
        out_ref[...] = out_ref[...] - (m + jnp.log(s))


def kernel(indices, emb_table, W0, b0, W1, b1, W2, b2):
    del W0, b0, W1, b1
    idx = indices.astype(jnp.int32)
    partials = _gather_sum(idx, emb_table)
    out = pl.pallas_call(
        _tc_matvec_lse,
        grid=(NB,),
        in_specs=[
            pl.BlockSpec((NWORK, EMB), lambda j: (0, 0)),
            pl.BlockSpec((BLK, EMB), lambda j: (j, 0)),
            pl.BlockSpec((NB, BLK), lambda j: (0, 0)),
        ],
        out_specs=pl.BlockSpec((NB, BLK), lambda j: (0, 0)),
        out_shape=jax.ShapeDtypeStruct((NB, BLK), jnp.float32),
        scratch_shapes=[
            pltpu.VMEM((1, BLK), jnp.float32),
            pltpu.VMEM((1, BLK), jnp.float32),
        ],
    )(partials, W2, b2.reshape(NB, BLK))
    return out.reshape(1, NCLS)
